# baseline (speedup 1.0000x reference)
import jax
import jax.numpy as jnp
from jax import lax
from jax.experimental import pallas as pl
from jax.experimental.pallas import tpu as pltpu, tpu_sc as plsc
import functools

_EMB = 128
_SU = 100.0
_SL = 0.0
_NC = 2
_NS = 16
_L = 16
_NW = _NC * _NS


_B = 1024
_S = 200
_ROWS = _B // _NW
_SLAB = _ROWS * _S
_JB = _S // _L
_TAIL = _S - _JB * _L


def _sc_body(x_hbm, sl_hbm, su_hbm, out_hbm, x_v, sl_v, su_v, ob_v, sem0, sem1):
    wid = lax.axis_index("s") * _NC + lax.axis_index("c")
    base = wid * _ROWS

    pltpu.sync_copy(x_hbm.at[pl.ds(base * _S, _SLAB)], x_v.at[pl.ds(0, _SLAB)])
    pltpu.sync_copy(sl_hbm, sl_v)
    pltpu.sync_copy(su_hbm, su_v)

    inv = 1.0 / (_SU - _SL)
    nchunk = _EMB // _L
    a = [sl_v[1, pl.ds(_L * c, _L)] for c in range(nchunk)]
    b = [(su_v[1, pl.ds(_L * c, _L)] - a[c]) * inv for c in range(nchunk)]
    sems = (sem0, sem1)

    def expand16(xvec, k, j0, nt):
        for t in range(nt):
            xv = jnp.full((_L,), xvec[t], jnp.float32)
            for c in range(nchunk):
                ob_v[k, j0 + t, pl.ds(_L * c, _L)] = a[c] + xv * b[c]

    def pair_body(p, carry):
        for k in range(2):
            r = 2 * p + k

            @pl.when(p > 0)
            def _wait():
                pltpu.make_async_copy(
                    ob_v.at[k], out_hbm.at[base + r - 2], sems[k]
                ).wait()

            roff = r * _S

            def jb_body(jb, c2):
                xvec = x_v[pl.ds(roff + jb * _L, _L)]
                expand16(xvec, k, jb * _L, _L)
                return c2

            lax.fori_loop(0, _JB, jb_body, 0)
            xvec = x_v[pl.ds(roff + _JB * _L, _L)]
            expand16(xvec, k, _JB * _L, _TAIL)

            pltpu.make_async_copy(ob_v.at[k], out_hbm.at[base + r], sems[k]).start()
        return carry

    lax.fori_loop(0, _ROWS // 2, pair_body, 0)
    pltpu.make_async_copy(ob_v.at[0], out_hbm.at[base + _ROWS - 2], sem0).wait()
    pltpu.make_async_copy(ob_v.at[1], out_hbm.at[base + _ROWS - 1], sem1).wait()


def kernel(mat2, emb_sl, emb_su):
    B, S = mat2.shape
    mesh = plsc.VectorSubcoreMesh(
        core_axis_name="c", subcore_axis_name="s",
        num_cores=_NC, num_subcores=_NS,
    )
    run = pl.kernel(
        _sc_body,
        out_type=jax.ShapeDtypeStruct((B, S, _EMB), jnp.float32),
        mesh=mesh,
        scratch_types=[
            pltpu.VMEM((_SLAB + _L,), jnp.float32),
            pltpu.VMEM((2, _EMB), jnp.float32),
            pltpu.VMEM((2, _EMB), jnp.float32),
            pltpu.VMEM((2, S, _EMB), jnp.float32),
            pltpu.SemaphoreType.DMA,
            pltpu.SemaphoreType.DMA,
        ],
    )
    return run(mat2.reshape(-1), emb_sl, emb_su)
